# Initial kernel scaffold; baseline (speedup 1.0000x reference)
import jax
import jax.numpy as jnp
from jax.experimental import pallas as pl


def kernel(feat, target, centers):
    raise NotImplementedError("write your pallas kernel here")


# speedup vs baseline: 1.5809x; 1.5809x over previous
import functools
import jax
import jax.numpy as jnp
from jax.experimental import pallas as pl
from jax.experimental.pallas import tpu as pltpu

_LAMDA = 0.5
_BLK = 512


def _loss_kernel(tgt_ref, x_ref, cen_ref, out_ref, acc_ref, *, nblk, batch):
    i = pl.program_id(0)

    @pl.when(i == 0)
    def _init():
        acc_ref[0, 0] = 0.0
        acc_ref[0, 1] = 0.0

    x = x_ref[...]
    tgt = tgt_ref[0, 0, :]
    blk, f = x.shape
    c = cen_ref.shape[0]

    m = jnp.max(x, axis=1, keepdims=True)
    lse = jnp.log(jnp.sum(jnp.exp(x - m), axis=1, keepdims=True)) + m

    cols_f = jax.lax.broadcasted_iota(jnp.int32, (blk, f), 1)
    mask_f = cols_f == tgt[:, None]
    picked_sum = jnp.sum(jnp.where(mask_f, x, 0.0))

    cols_c = jax.lax.broadcasted_iota(jnp.int32, (blk, c), 1)
    onehot = (cols_c == tgt[:, None]).astype(jnp.bfloat16)
    cb = jax.lax.dot_general(
        onehot, cen_ref[...],
        (((1,), (0,)), ((), ())),
        preferred_element_type=jnp.float32,
    )
    diff = cb - x
    center_part = jnp.sum(diff * diff)

    acc_ref[0, 0] += jnp.sum(lse) - picked_sum
    acc_ref[0, 1] += center_part

    @pl.when(i == nblk - 1)
    def _fin():
        out_ref[0, 0] = (acc_ref[0, 0] / batch
                         + _LAMDA * acc_ref[0, 1] / 2.0 / batch)


def kernel(feat, target, centers):
    batch, f = feat.shape
    c = centers.shape[0]
    nblk = batch // _BLK
    tgt3 = target.astype(jnp.int32).reshape(nblk, 1, _BLK)
    cen_bf = centers.astype(jnp.bfloat16)

    out = pl.pallas_call(
        functools.partial(_loss_kernel, nblk=nblk, batch=batch),
        grid=(nblk,),
        in_specs=[
            pl.BlockSpec((1, 1, _BLK), lambda i: (i, 0, 0)),
            pl.BlockSpec((_BLK, f), lambda i: (i, 0)),
            pl.BlockSpec((c, f), lambda i: (0, 0)),
        ],
        out_specs=pl.BlockSpec(memory_space=pltpu.SMEM),
        out_shape=jax.ShapeDtypeStruct((1, 1), jnp.float32),
        scratch_shapes=[pltpu.SMEM((1, 2), jnp.float32)],
    )(tgt3, feat, cen_bf)
    return out[0, 0]
